# baseline (speedup 1.0000x reference)
import functools

import jax
import jax.numpy as jnp
from jax import lax
from jax.experimental import pallas as pl
from jax.experimental.pallas import tpu as pltpu
from jax.experimental.pallas import tpu_sc as plsc

_H, _W, _B = 176, 200, 4
_C = 128
_NPTS = 65536
_NC, _NS = 2, 16
_PTS = _NPTS // _NS
_GRID_ROWS = _B * _W * _H
_SLICE_ROWS = 7168
_NSLICE = -(-_GRID_ROWS // _SLICE_ROWS)
_ROUNDS = _NSLICE // _NC
_ACC_ROWS = _SLICE_ROWS + 8
_G = 128
_NCHUNK = _PTS // _G
_OUT_ROWS = _NPTS + 16
_ZCOPIES = _SLICE_ROWS // _NS // _G
_STG = 1024
_PIDB = 17


def _sc_cell_sums(wf, idx_t):
    mesh = plsc.VectorSubcoreMesh(core_axis_name="c", subcore_axis_name="s")

    @functools.partial(
        pl.kernel,
        mesh=mesh,
        out_type=jax.ShapeDtypeStruct((_OUT_ROWS, _C), jnp.float32),
        compiler_params=pltpu.CompilerParams(needs_layout_passes=False),
        scratch_types=[
            pltpu.VMEM((_PTS,), jnp.int32),
            pltpu.VMEM((_STG,), jnp.int32),
            pltpu.VMEM((_NCHUNK, _G), jnp.int32),
            pltpu.VMEM((1, _G), jnp.int32),
            pltpu.VMEM((1, _G), jnp.int32),
            pltpu.VMEM((1, _G), jnp.int32),
            pltpu.VMEM((1, _G), jnp.int32),
            pltpu.VMEM((1, _G), jnp.int32),
            pltpu.VMEM((1, _G), jnp.int32),
            pltpu.VMEM((_G, _C), jnp.float32),
            pltpu.VMEM((_G, _C), jnp.float32),
            pltpu.VMEM((_G, _C), jnp.float32),
            pltpu.VMEM_SHARED((_ACC_ROWS, _C), jnp.float32),
            pltpu.SemaphoreType.DMA,
            pltpu.SemaphoreType.DMA,
            pltpu.SemaphoreType.DMA,
            pltpu.SemaphoreType.DMA,
            pltpu.SemaphoreType.DMA,
        ],
    )
    def k(wf_hbm, idxt_hbm, out_hbm,
          fi_v, stg_v, q_l, cc_a, pg_a, ps_a, cc_b, pg_b, ps_b,
          rows_a, rows_b, zero_v, acc, gsem_a, gsem_b, zsem, osem_a, osem_b):
        c = lax.axis_index("c")
        s = lax.axis_index("s")
        base = s * _PTS

        pltpu.sync_copy(idxt_hbm.at[pl.ds(_NPTS + base, _PTS)], fi_v)
        for row, mult in ((2, _H), (0, _H * _W)):
            for j in range(_PTS // _STG):
                pltpu.sync_copy(
                    idxt_hbm.at[pl.ds(row * _NPTS + base + j * _STG, _STG)],
                    stg_v)

                def acc_body(i, carry, _j=j, _m=mult):
                    sl16 = pl.ds(_j * _STG + i * 16, 16)
                    fi_v[sl16] = fi_v[sl16] + stg_v[pl.ds(i * 16, 16)] * _m
                    return carry
                lax.fori_loop(0, _STG // 16, acc_body, 0)

        zvec = jnp.zeros((16,), jnp.float32)

        def z_body(i, carry):
            zero_v[i // 8, pl.ds((i % 8) * 16, 16)] = zvec
            return carry
        lax.fori_loop(0, _G * (_C // 16), z_body, 0)

        lane = lax.iota(jnp.int32, 16)
        cell_pad = _SLICE_ROWS + (s % 8)
        pad_packed = jnp.full((16,), cell_pad << _PIDB, jnp.int32)
        pidg_pad = s
        pids_pad = _NPTS + s
        zbase = s * (_SLICE_ROWS // _NS)

        def unpack(ch, cc, pg, ps):
            def u_body(i, carry):
                pk = q_l[ch, pl.ds(i * 16, 16)]
                cell16 = lax.shift_right_arithmetic(pk, _PIDB)
                pid16 = pk & ((1 << _PIDB) - 1)
                ispad = cell16 >= _SLICE_ROWS
                cc[0, pl.ds(i * 16, 16)] = cell16
                pg[0, pl.ds(i * 16, 16)] = jnp.where(ispad, pidg_pad, pid16)
                ps[0, pl.ds(i * 16, 16)] = jnp.where(ispad, pids_pad, pid16)
                return carry
            lax.fori_loop(0, _G // 16, u_body, 0)

        def one_pass(r, carry):
            lo = (_NC * r + c) * _SLICE_ROWS
            hi = lo + _SLICE_ROWS

            zh = [pltpu.async_copy(zero_v, acc.at[pl.ds(zbase + j * _G, _G)],
                                   zsem)
                  for j in range(_ZCOPIES)]
            zrem = _SLICE_ROWS // _NS - _ZCOPIES * _G
            if zrem:
                zh.append(pltpu.async_copy(
                    zero_v.at[pl.ds(0, zrem)],
                    acc.at[pl.ds(zbase + _ZCOPIES * _G, zrem)], zsem))

            def c_body(i, cnt):
                fi16 = fi_v[pl.ds(i * 16, 16)]
                m = (fi16 >= lo) & (fi16 < hi)
                mi = m.astype(jnp.int32)
                pos = cnt + jnp.cumsum(mi) - 1
                pid16 = base + i * 16 + lane
                pk = lax.shift_left(fi16 - lo, _PIDB) | pid16
                plsc.store_scatter(q_l, [pos >> 7, pos & (_G - 1)], pk,
                                   mask=m)
                return cnt + jnp.sum(mi)
            cnt = lax.fori_loop(0, _PTS // 16, c_body, 0)
            nch = (cnt + _G - 1) // _G

            def t_body(g, carry2):
                sl16 = pl.ds((g % (_G // 16)) * 16, 16)
                old = q_l[g // (_G // 16), sl16]
                idx16 = g * 16 + lane
                q_l[g // (_G // 16), sl16] = jnp.where(idx16 >= cnt, pad_packed, old)
                return carry2
            lax.fori_loop(cnt >> 4, (nch * _G) >> 4, t_body, 0)

            for h in zh:
                h.wait()
            plsc.subcore_barrier()

            @pl.when(nch > 0)
            def _():
                unpack(0, cc_a, pg_a, ps_a)
                pltpu.async_copy(wf_hbm.at[pg_a.at[0]], rows_a, gsem_a)

            def add_step(kk, carry2):
                ch0 = 2 * kk
                ch1 = ch0 + 1
                ch2 = ch0 + 2

                @pl.when(ch1 < nch)
                def _():
                    unpack(ch1, cc_b, pg_b, ps_b)
                    pltpu.async_copy(wf_hbm.at[pg_b.at[0]], rows_b, gsem_b)
                pltpu.make_async_copy(wf_hbm.at[pg_a.at[0]], rows_a,
                                      gsem_a).wait()
                pltpu.sync_copy(rows_a, acc.at[cc_a.at[0]], add=True)

                @pl.when(ch2 < nch)
                def _():
                    unpack(ch2, cc_a, pg_a, ps_a)
                    pltpu.async_copy(wf_hbm.at[pg_a.at[0]], rows_a, gsem_a)

                @pl.when(ch1 < nch)
                def _():
                    pltpu.make_async_copy(wf_hbm.at[pg_b.at[0]], rows_b,
                                          gsem_b).wait()
                    pltpu.sync_copy(rows_b, acc.at[cc_b.at[0]], add=True)
                return carry2
            lax.fori_loop(0, (nch + 1) >> 1, add_step, 0)
            plsc.subcore_barrier()

            def out_step(kk, carry2):
                ch0 = 2 * kk
                ch1 = ch0 + 1

                @pl.when(kk > 0)
                def _():
                    pltpu.make_async_copy(rows_a, out_hbm.at[ps_a.at[0]],
                                          osem_a).wait()
                unpack(ch0, cc_a, pg_a, ps_a)
                pltpu.sync_copy(acc.at[cc_a.at[0]], rows_a)
                pltpu.async_copy(rows_a, out_hbm.at[ps_a.at[0]], osem_a)

                @pl.when(ch1 < nch)
                def _():
                    @pl.when(kk > 0)
                    def _():
                        pltpu.make_async_copy(rows_b, out_hbm.at[ps_b.at[0]],
                                              osem_b).wait()
                    unpack(ch1, cc_b, pg_b, ps_b)
                    pltpu.sync_copy(acc.at[cc_b.at[0]], rows_b)
                    pltpu.async_copy(rows_b, out_hbm.at[ps_b.at[0]], osem_b)
                return carry2
            nsteps = (nch + 1) >> 1
            lax.fori_loop(0, nsteps, out_step, 0)

            @pl.when(nch > 0)
            def _():
                pltpu.make_async_copy(rows_a, out_hbm.at[ps_a.at[0]],
                                      osem_a).wait()

            @pl.when(nch > 1)
            def _():
                pltpu.make_async_copy(rows_b, out_hbm.at[ps_b.at[0]],
                                      osem_b).wait()
            plsc.subcore_barrier()
            return carry

        lax.fori_loop(0, _ROUNDS, one_pass, 0)

    return k(wf, idx_t)


_BLK = 16384


def _ln_mm_body(x_ref, sc_ref, bi_ref, w_ref, o_ref):
    x = x_ref[...]
    mu = jnp.mean(x, axis=1, keepdims=True)
    xc = x - mu
    var = jnp.mean(xc * xc, axis=1, keepdims=True)
    y = xc * lax.rsqrt(var + 1e-5) * sc_ref[...] + bi_ref[...]
    o_ref[...] = jnp.dot(y, w_ref[...], preferred_element_type=jnp.float32)


def _tc_ln_matmul(cellsum, norm_scale, norm_bias, w_t):
    grid = _NPTS // _BLK
    return pl.pallas_call(
        _ln_mm_body,
        grid=(grid,),
        in_specs=[
            pl.BlockSpec((_BLK, _C), lambda i: (i, 0)),
            pl.BlockSpec((1, _C), lambda i: (0, 0)),
            pl.BlockSpec((1, _C), lambda i: (0, 0)),
            pl.BlockSpec((_C, _C), lambda i: (0, 0)),
        ],
        out_specs=pl.BlockSpec((_BLK, _C), lambda i: (i, 0)),
        out_shape=jax.ShapeDtypeStruct((_NPTS, _C), jnp.float32),
    )(cellsum, norm_scale.reshape(1, _C), norm_bias.reshape(1, _C), w_t)


def kernel(window_features, window_indices, norm_scale, norm_bias, linear_w):
    bw, n, cdim = window_features.shape
    wf = window_features.reshape(-1, cdim)
    idx_t = window_indices.reshape(-1, 3).T.reshape(-1)
    cellsum = _sc_cell_sums(wf, idx_t)
    out = _tc_ln_matmul(cellsum, norm_scale, norm_bias, linear_w.T)
    return out.reshape(bw, n, cdim)


# speedup vs baseline: 1.2092x; 1.2092x over previous
import functools

import jax
import jax.numpy as jnp
from jax import lax
from jax.experimental import pallas as pl
from jax.experimental.pallas import tpu as pltpu
from jax.experimental.pallas import tpu_sc as plsc

_H, _W, _B = 176, 200, 4
_C = 128
_NPTS = 65536
_NC, _NS = 2, 16
_PTS = _NPTS // _NS
_GRID_ROWS = _B * _W * _H
_SLICE_ROWS = 10240
_NSLICE = -(-_GRID_ROWS // _SLICE_ROWS)
_ROUNDS = _NSLICE // _NC
_ACC_ROWS = _SLICE_ROWS + 8
_G = 64
_NCHUNK = _PTS // _G
_OUT_ROWS = _NPTS + 16
_ZCOPIES = _SLICE_ROWS // _NS // _G
_STG = 1024
_PIDB = 17


def _sc_cell_sums(wf, idx_t):
    mesh = plsc.VectorSubcoreMesh(core_axis_name="c", subcore_axis_name="s")

    @functools.partial(
        pl.kernel,
        mesh=mesh,
        out_type=jax.ShapeDtypeStruct((_OUT_ROWS, _C), jnp.float32),
        compiler_params=pltpu.CompilerParams(needs_layout_passes=False),
        scratch_types=[
            pltpu.VMEM((_PTS,), jnp.int32),
            pltpu.VMEM((_STG,), jnp.int32),
            pltpu.VMEM((_NCHUNK, _G), jnp.int32),
            pltpu.VMEM((1, _G), jnp.int32),
            pltpu.VMEM((1, _G), jnp.int32),
            pltpu.VMEM((1, _G), jnp.int32),
            pltpu.VMEM((1, _G), jnp.int32),
            pltpu.VMEM((1, _G), jnp.int32),
            pltpu.VMEM((1, _G), jnp.int32),
            pltpu.VMEM((_G, _C), jnp.float32),
            pltpu.VMEM((_G, _C), jnp.float32),
            pltpu.VMEM((_G, _C), jnp.float32),
            pltpu.VMEM_SHARED((_ACC_ROWS, _C), jnp.float32),
            pltpu.SemaphoreType.DMA,
            pltpu.SemaphoreType.DMA,
            pltpu.SemaphoreType.DMA,
            pltpu.SemaphoreType.DMA,
            pltpu.SemaphoreType.DMA,
        ],
    )
    def k(wf_hbm, idxt_hbm, out_hbm,
          fi_v, stg_v, q_l, cc_a, pg_a, ps_a, cc_b, pg_b, ps_b,
          rows_a, rows_b, zero_v, acc, gsem_a, gsem_b, zsem, osem_a, osem_b):
        c = lax.axis_index("c")
        s = lax.axis_index("s")
        base = s * _PTS

        pltpu.sync_copy(idxt_hbm.at[pl.ds(_NPTS + base, _PTS)], fi_v)
        for row, mult in ((2, _H), (0, _H * _W)):
            for j in range(_PTS // _STG):
                pltpu.sync_copy(
                    idxt_hbm.at[pl.ds(row * _NPTS + base + j * _STG, _STG)],
                    stg_v)

                def acc_body(i, carry, _j=j, _m=mult):
                    sl16 = pl.ds(_j * _STG + i * 16, 16)
                    fi_v[sl16] = fi_v[sl16] + stg_v[pl.ds(i * 16, 16)] * _m
                    return carry
                lax.fori_loop(0, _STG // 16, acc_body, 0)

        zvec = jnp.zeros((16,), jnp.float32)

        def z_body(i, carry):
            zero_v[i // 8, pl.ds((i % 8) * 16, 16)] = zvec
            return carry
        lax.fori_loop(0, _G * (_C // 16), z_body, 0)

        lane = lax.iota(jnp.int32, 16)
        cell_pad = _SLICE_ROWS + (s % 8)
        pad_packed = jnp.full((16,), cell_pad << _PIDB, jnp.int32)
        pidg_pad = s
        pids_pad = _NPTS + s
        zbase = s * (_SLICE_ROWS // _NS)

        def unpack(ch, cc, pg, ps):
            def u_body(i, carry):
                pk = q_l[ch, pl.ds(i * 16, 16)]
                cell16 = lax.shift_right_arithmetic(pk, _PIDB)
                pid16 = pk & ((1 << _PIDB) - 1)
                ispad = cell16 >= _SLICE_ROWS
                cc[0, pl.ds(i * 16, 16)] = cell16
                pg[0, pl.ds(i * 16, 16)] = jnp.where(ispad, pidg_pad, pid16)
                ps[0, pl.ds(i * 16, 16)] = jnp.where(ispad, pids_pad, pid16)
                return carry
            lax.fori_loop(0, _G // 16, u_body, 0)

        def one_pass(r, carry):
            lo = (_NC * r + c) * _SLICE_ROWS
            hi = lo + _SLICE_ROWS

            zh = [pltpu.async_copy(zero_v, acc.at[pl.ds(zbase + j * _G, _G)],
                                   zsem)
                  for j in range(_ZCOPIES)]

            def c_body(i, cnt):
                fi16 = fi_v[pl.ds(i * 16, 16)]
                m = (fi16 >= lo) & (fi16 < hi)
                mi = m.astype(jnp.int32)
                pos = cnt + jnp.cumsum(mi) - 1
                pid16 = base + i * 16 + lane
                pk = lax.shift_left(fi16 - lo, _PIDB) | pid16
                plsc.store_scatter(q_l, [pos >> 6, pos & (_G - 1)], pk,
                                   mask=m)
                return cnt + jnp.sum(mi)
            cnt = lax.fori_loop(0, _PTS // 16, c_body, 0)
            nch = (cnt + _G - 1) // _G

            def t_body(g, carry2):
                sl16 = pl.ds((g % (_G // 16)) * 16, 16)
                old = q_l[g >> 2, sl16]
                idx16 = g * 16 + lane
                q_l[g >> 2, sl16] = jnp.where(idx16 >= cnt, pad_packed, old)
                return carry2
            lax.fori_loop(cnt >> 4, (nch * _G) >> 4, t_body, 0)

            @pl.when(nch > 0)
            def _():
                unpack(0, cc_a, pg_a, ps_a)
                pltpu.async_copy(wf_hbm.at[pg_a.at[0]], rows_a, gsem_a)

            @pl.when(nch > 1)
            def _():
                unpack(1, cc_b, pg_b, ps_b)
                pltpu.async_copy(wf_hbm.at[pg_b.at[0]], rows_b, gsem_b)

            for h in zh:
                h.wait()
            plsc.subcore_barrier()

            def add_step(kk, carry2):
                ch0 = 2 * kk
                ch1 = ch0 + 1

                pltpu.make_async_copy(wf_hbm.at[pg_a.at[0]], rows_a,
                                      gsem_a).wait()
                pltpu.sync_copy(rows_a, acc.at[cc_a.at[0]], add=True)

                @pl.when(ch0 + 2 < nch)
                def _():
                    unpack(ch0 + 2, cc_a, pg_a, ps_a)
                    pltpu.async_copy(wf_hbm.at[pg_a.at[0]], rows_a, gsem_a)

                @pl.when(ch1 < nch)
                def _():
                    pltpu.make_async_copy(wf_hbm.at[pg_b.at[0]], rows_b,
                                          gsem_b).wait()
                    pltpu.sync_copy(rows_b, acc.at[cc_b.at[0]], add=True)

                    @pl.when(ch1 + 2 < nch)
                    def _():
                        unpack(ch1 + 2, cc_b, pg_b, ps_b)
                        pltpu.async_copy(wf_hbm.at[pg_b.at[0]], rows_b,
                                         gsem_b)
                return carry2
            lax.fori_loop(0, (nch + 1) >> 1, add_step, 0)
            plsc.subcore_barrier()

            def out_step(kk, carry2):
                ch0 = 2 * kk
                ch1 = ch0 + 1

                @pl.when(kk > 0)
                def _():
                    pltpu.make_async_copy(rows_a, out_hbm.at[ps_a.at[0]],
                                          osem_a).wait()
                unpack(ch0, cc_a, pg_a, ps_a)
                pltpu.sync_copy(acc.at[cc_a.at[0]], rows_a)
                pltpu.async_copy(rows_a, out_hbm.at[ps_a.at[0]], osem_a)

                @pl.when(ch1 < nch)
                def _():
                    @pl.when(kk > 0)
                    def _():
                        pltpu.make_async_copy(rows_b, out_hbm.at[ps_b.at[0]],
                                              osem_b).wait()
                    unpack(ch1, cc_b, pg_b, ps_b)
                    pltpu.sync_copy(acc.at[cc_b.at[0]], rows_b)
                    pltpu.async_copy(rows_b, out_hbm.at[ps_b.at[0]], osem_b)
                return carry2
            nsteps = (nch + 1) >> 1
            lax.fori_loop(0, nsteps, out_step, 0)

            @pl.when(nch > 0)
            def _():
                pltpu.make_async_copy(rows_a, out_hbm.at[ps_a.at[0]],
                                      osem_a).wait()

            @pl.when(nch > 1)
            def _():
                pltpu.make_async_copy(rows_b, out_hbm.at[ps_b.at[0]],
                                      osem_b).wait()
            plsc.subcore_barrier()
            return carry

        lax.fori_loop(0, _ROUNDS, one_pass, 0)

    return k(wf, idx_t)


_BLK = 16384


def _ln_mm_body(x_ref, sc_ref, bi_ref, w_ref, o_ref):
    x = x_ref[...]
    mu = jnp.mean(x, axis=1, keepdims=True)
    xc = x - mu
    var = jnp.mean(xc * xc, axis=1, keepdims=True)
    y = xc * lax.rsqrt(var + 1e-5) * sc_ref[...] + bi_ref[...]
    o_ref[...] = jnp.dot(y, w_ref[...], preferred_element_type=jnp.float32)


def _tc_ln_matmul(cellsum, norm_scale, norm_bias, w_t):
    grid = _NPTS // _BLK
    return pl.pallas_call(
        _ln_mm_body,
        grid=(grid,),
        in_specs=[
            pl.BlockSpec((_BLK, _C), lambda i: (i, 0)),
            pl.BlockSpec((1, _C), lambda i: (0, 0)),
            pl.BlockSpec((1, _C), lambda i: (0, 0)),
            pl.BlockSpec((_C, _C), lambda i: (0, 0)),
        ],
        out_specs=pl.BlockSpec((_BLK, _C), lambda i: (i, 0)),
        out_shape=jax.ShapeDtypeStruct((_NPTS, _C), jnp.float32),
    )(cellsum, norm_scale.reshape(1, _C), norm_bias.reshape(1, _C), w_t)


def kernel(window_features, window_indices, norm_scale, norm_bias, linear_w):
    bw, n, cdim = window_features.shape
    wf = window_features.reshape(-1, cdim)
    idx_t = window_indices.reshape(-1, 3).T.reshape(-1)
    cellsum = _sc_cell_sums(wf, idx_t)
    out = _tc_ln_matmul(cellsum, norm_scale, norm_bias, linear_w.T)
    return out.reshape(bw, n, cdim)
